# Initial kernel scaffold; baseline (speedup 1.0000x reference)
import jax
import jax.numpy as jnp
from jax.experimental import pallas as pl


def kernel(indices, img, features, Wk, bk, Wq, bq):
    raise NotImplementedError("write your pallas kernel here")


# speedup vs baseline: 44.6431x; 44.6431x over previous
import functools
import jax
import jax.numpy as jnp
from jax import lax
from jax.experimental import pallas as pl
from jax.experimental.pallas import tpu as pltpu
from jax.experimental.pallas import tpu_sc as plsc

_B, _N, _K = 4, 4096, 64
_LAT, _D = 128, 32
_ITEMS = _B * _N * _K
_ROWS = _B * _N
_NW = 32
_PER_W = _ITEMS // _NW
_CHUNK = 512
_NCHUNK = _PER_W // _CHUNK
_L = 16
_SCALE = float(_D) ** -0.5


def _kq_body(f_ref, wk_ref, bk_ref, wq_ref, bq_ref, k_ref, q_ref):
    f = f_ref[...]
    k_ref[...] = (
        jnp.dot(f, wk_ref[...], preferred_element_type=jnp.float32,
                precision=lax.Precision.HIGHEST) + bk_ref[...]
    )
    q_ref[...] = (
        jnp.dot(f, wq_ref[...], preferred_element_type=jnp.float32,
                precision=lax.Precision.HIGHEST) + bq_ref[...]
    )


def _make_tables(features2d, Wk, bk2d, Wq, bq2d):
    R = 2048
    grid = (_ROWS // R,)
    return pl.pallas_call(
        _kq_body,
        grid=grid,
        in_specs=[
            pl.BlockSpec((R, _LAT), lambda i: (i, 0)),
            pl.BlockSpec((_LAT, _D), lambda i: (0, 0)),
            pl.BlockSpec((1, _D), lambda i: (0, 0)),
            pl.BlockSpec((_LAT, _D), lambda i: (0, 0)),
            pl.BlockSpec((1, _D), lambda i: (0, 0)),
        ],
        out_specs=[
            pl.BlockSpec((R, _D), lambda i: (i, 0)),
            pl.BlockSpec((R, _D), lambda i: (i, 0)),
        ],
        out_shape=[
            jax.ShapeDtypeStruct((_ROWS, _D), jnp.float32),
            jax.ShapeDtypeStruct((_ROWS, _D), jnp.float32),
        ],
    )(features2d, Wk, bk2d, Wq, bq2d)


def _affinity_body(ks_hbm, qs_hbm, xidx_hbm, yidx_hbm, out_hbm,
                   xidx_v, yidx_v, xrows_v, yrows_v, prod_v, out_v,
                   sem1, sem2):
    wid = lax.axis_index("s") * 2 + lax.axis_index("c")
    wbase = wid * _PER_W
    row_off = (wbase // (_N * _K)) * _N
    scatter_base = lax.iota(jnp.int32, _L) * _CHUNK

    def chunk_body(c, _):
        base = wbase + c * _CHUNK
        pltpu.sync_copy(xidx_hbm.at[pl.ds(base, _CHUNK)], xidx_v)
        pltpu.sync_copy(yidx_hbm.at[pl.ds(base, _CHUNK)], yidx_v)

        def rebase(i, _):
            s = pl.ds(i * _L, _L)
            xidx_v[s] = xidx_v[s] + row_off
            yidx_v[s] = yidx_v[s] + row_off
            return 0

        lax.fori_loop(0, _CHUNK // _L, rebase, 0)

        cx = pltpu.async_copy(ks_hbm.at[xidx_v], xrows_v, sem1)
        cy = pltpu.async_copy(qs_hbm.at[yidx_v], yrows_v, sem2)
        cx.wait()
        cy.wait()

        def item(i, _):
            x0 = xrows_v[i, pl.ds(0, _L)]
            x1 = xrows_v[i, pl.ds(_L, _L)]
            y0 = yrows_v[i, pl.ds(0, _L)]
            y1 = yrows_v[i, pl.ds(_L, _L)]
            p = x0 * y0 + x1 * y1
            plsc.store_scatter(prod_v, [scatter_base + i], p)
            return 0

        lax.fori_loop(0, _CHUNK, item, 0)

        def group(g, _):
            t = prod_v[pl.ds(g * _L, _L)]
            for s in range(1, _L):
                t = t + prod_v[pl.ds(s * _CHUNK + g * _L, _L)]
            out_v[pl.ds(g * _L, _L)] = t * _SCALE
            return 0

        lax.fori_loop(0, _CHUNK // _L, group, 0)
        pltpu.sync_copy(out_v, out_hbm.at[pl.ds(base, _CHUNK)])
        return 0

    lax.fori_loop(0, _NCHUNK, chunk_body, 0)


@functools.partial(
    pl.kernel,
    out_type=jax.ShapeDtypeStruct((_ITEMS,), jnp.float32),
    mesh=plsc.VectorSubcoreMesh(core_axis_name="c", subcore_axis_name="s"),
    scratch_types=[
        pltpu.VMEM((_CHUNK,), jnp.int32),
        pltpu.VMEM((_CHUNK,), jnp.int32),
        pltpu.VMEM((_CHUNK, _D), jnp.float32),
        pltpu.VMEM((_CHUNK, _D), jnp.float32),
        pltpu.VMEM((_L * _CHUNK,), jnp.float32),
        pltpu.VMEM((_CHUNK,), jnp.float32),
        pltpu.SemaphoreType.DMA,
        pltpu.SemaphoreType.DMA,
    ],
    compiler_params=pltpu.CompilerParams(
        needs_layout_passes=False, use_tc_tiling_on_sc=False),
)
def _affinity(ks_hbm, qs_hbm, xidx_hbm, yidx_hbm, out_hbm,
              xidx_v, yidx_v, xrows_v, yrows_v, prod_v, out_v, sem1, sem2):
    _affinity_body(ks_hbm, qs_hbm, xidx_hbm, yidx_hbm, out_hbm,
                   xidx_v, yidx_v, xrows_v, yrows_v, prod_v, out_v,
                   sem1, sem2)


@jax.jit
def kernel(indices, img, features, Wk, bk, Wq, bq):
    del img
    ks, qs = _make_tables(
        features.reshape(_ROWS, _LAT), Wk, bk.reshape(1, _D),
        Wq, bq.reshape(1, _D))
    x_idx = indices[1].reshape(_ITEMS)
    y_idx = indices[2].reshape(_ITEMS)
    out = _affinity(ks, qs, x_idx, y_idx)
    return out.reshape(_B, _N, _K)
